# baseline (speedup 1.0000x reference)
import functools

import numpy as np

import jax
import jax.numpy as jnp
from jax import lax
from jax.experimental import pallas as pl
from jax.experimental.pallas import tpu as pltpu
from jax.experimental.pallas import tpu_sc as plsc

N = 50000
E = 800000
D = 64
PACK = 4
EP = E // PACK
DP = D * PACK
BM = 1000

HALF = N // 2
ACC_ROWS = 25088
CHUNK = 128
NCHUNK = E // CHUNK
NSUB = 16
CPW = 392
NCPAD = NSUB * CPW

_PERM = np.zeros(D, dtype=np.int32)
for _g in (0, 1):
    for _i in range(16):
        _PERM[32 * _g + 2 * _i] = 32 * _g + _i
        _PERM[32 * _g + 2 * _i + 1] = 32 * _g + 16 + _i


_LP = (1.472065010887924e-06, 0.9998476974962351, -0.49737321615793884,
       0.3157473167579205, -0.19035433673298097, 0.08269123711134978,
       -0.017414077524237504)


def _mlp_body(rbf_ref, w1_ref, b1_ref, w2_ref, b2_ref, out_ref):
    a = jnp.dot(rbf_ref[...].astype(jnp.bfloat16), w1_ref[...],
                preferred_element_type=jnp.float32)
    a = a + b1_ref[...]
    u = jnp.exp(-0.5 * jnp.abs(a))
    p = _LP[6]
    for k in (5, 4, 3, 2, 1, 0):
        p = p * u + _LP[k]
    a = jnp.maximum(a, 0.0) + 2.0 * p
    o = jnp.dot(a.astype(jnp.bfloat16), w2_ref[...],
                preferred_element_type=jnp.float32)
    out_ref[...] = o + b2_ref[...]


def _edge_mlp(rbf4, w1bd, b1t, w2bd, b2t):
    return pl.pallas_call(
        _mlp_body,
        grid=(EP // BM,),
        in_specs=[
            pl.BlockSpec((BM, DP), lambda i: (i, 0)),
            pl.BlockSpec((DP, DP), lambda i: (0, 0)),
            pl.BlockSpec((1, DP), lambda i: (0, 0)),
            pl.BlockSpec((DP, DP), lambda i: (0, 0)),
            pl.BlockSpec((1, DP), lambda i: (0, 0)),
        ],
        out_specs=pl.BlockSpec((BM, DP), lambda i: (i, 0)),
        out_shape=jax.ShapeDtypeStruct((EP, DP), jnp.float32),
    )(rbf4, w1bd, b1t, w2bd, b2t)


def _mul_rows(h_ref, x_ref):
    def _row(r, carry):
        for g in range(2):
            v = x_ref[r, pl.ds(g * 16, 16)]
            a = (v >> 16).astype(jnp.float32)
            b = ((v << 16) >> 16).astype(jnp.float32)
            s0 = pl.ds(g * 32, 16)
            s1 = pl.ds(g * 32 + 16, 16)
            h_ref[r, s0] = h_ref[r, s0] * b
            h_ref[r, s1] = h_ref[r, s1] * a
        return carry
    lax.fori_loop(0, CHUNK, _row, 0)


def _sc_body(x_hbm, h_hbm, src_hbm, dst_hbm, out_hbm,
             src_sv, dst_sv, x0, x1, h0, h1, acc,
             sga, sgb, sha, shb):
    c = lax.axis_index("c")
    s = lax.axis_index("s")
    w = s
    lo = c * HALF

    def _zrow(i, carry):
        for c4 in range(4):
            h0[i, pl.ds(c4 * 16, 16)] = jnp.zeros((16,), jnp.float32)
        return carry
    lax.fori_loop(0, CHUNK, _zrow, 0)
    zbase = s * (ACC_ROWS // NSUB)
    for k in range(ACC_ROWS // NSUB // CHUNK):
        pltpu.sync_copy(h0.at[pl.ds(0, CHUNK)],
                        acc.at[pl.ds(zbase + k * CHUNK, CHUNK)])
    pltpu.sync_copy(h0.at[pl.ds(0, 32)], acc.at[pl.ds(zbase + 1536, 32)])
    plsc.subcore_barrier()

    def _pair(jj, carry):
        sup = jj % 4
        base_t = w * CPW + jj * 2

        @pl.when(sup == 0)
        def _load_super():
            pltpu.sync_copy(src_hbm.at[pl.ds(base_t, 8)], src_sv)
            pltpu.sync_copy(dst_hbm.at[pl.ds(base_t, 8)], dst_sv)
            for r in range(8):
                lo_r = jnp.where(base_t + r < NCHUNK, lo, N + D)
                for c8 in range(8):
                    sl = pl.ds(c8 * 16, 16)
                    dv = dst_sv[r, sl]
                    m = (dv >= lo_r) & (dv < lo_r + HALF)
                    dst_sv[r, sl] = jnp.where(m, dv - lo_r, HALF)

        ra = sup * 2
        ta = jnp.minimum(base_t, NCHUNK - 1)
        tb = jnp.minimum(base_t + 1, NCHUNK - 1)
        cpxa = pltpu.async_copy(x_hbm.at[src_sv.at[ra]], x0, sga)
        cpha = pltpu.async_copy(h_hbm.at[pl.ds(ta * CHUNK, CHUNK)], h0, sha)
        cpxb = pltpu.async_copy(x_hbm.at[src_sv.at[ra + 1]], x1, sgb)
        cphb = pltpu.async_copy(h_hbm.at[pl.ds(tb * CHUNK, CHUNK)], h1, shb)
        cpha.wait()
        cpxa.wait()
        _mul_rows(h0, x0)
        pltpu.sync_copy(h0, acc.at[dst_sv.at[ra]], add=True)
        cphb.wait()
        cpxb.wait()
        _mul_rows(h1, x1)
        pltpu.sync_copy(h1, acc.at[dst_sv.at[ra + 1]], add=True)
        return carry
    lax.fori_loop(0, CPW // 2, _pair, 0)

    plsc.subcore_barrier()
    obase = lo + s * 1560
    pltpu.sync_copy(acc.at[pl.ds(s * 1560, 1560)],
                    out_hbm.at[pl.ds(obase, 1560)])

    @pl.when(s == NSUB - 1)
    def _tail():
        pltpu.sync_copy(acc.at[pl.ds(24960, 40)],
                        out_hbm.at[pl.ds(lo + 24960, 40)])


_sc_kernel = functools.partial(
    pl.kernel,
    mesh=plsc.VectorSubcoreMesh(core_axis_name="c", subcore_axis_name="s"),
    compiler_params=pltpu.CompilerParams(use_tc_tiling_on_sc=False),
    out_type=jax.ShapeDtypeStruct((N, D), jnp.float32),
    scratch_types=[
        pltpu.VMEM((8, 128), jnp.int32),
        pltpu.VMEM((8, 128), jnp.int32),
        pltpu.VMEM((CHUNK, D // 2), jnp.int32),
        pltpu.VMEM((CHUNK, D // 2), jnp.int32),
        pltpu.VMEM((CHUNK, D), jnp.float32),
        pltpu.VMEM((CHUNK, D), jnp.float32),
        pltpu.VMEM_SHARED((ACC_ROWS, D), jnp.float32),
        pltpu.SemaphoreType.DMA,
        pltpu.SemaphoreType.DMA,
        pltpu.SemaphoreType.DMA,
        pltpu.SemaphoreType.DMA,
    ],
)(_sc_body)


def kernel(x, rbf, edge_index, W1, b1, W2, b2):
    src = edge_index[0].astype(jnp.int32).reshape(NCHUNK, 128)
    dst = edge_index[1].astype(jnp.int32).reshape(NCHUNK, 128)
    pad = ((0, NCPAD - NCHUNK), (0, 0))
    src = jnp.pad(src, pad)
    dst = jnp.pad(dst, pad)
    xq = jnp.clip(jnp.round(x * 4096.0), -32768.0, 32767.0).astype(jnp.int16)
    xp = lax.bitcast_convert_type(
        xq[:, jnp.asarray(_PERM)].reshape(N, D // 2, 2), jnp.int32)
    eye4 = jnp.eye(PACK, dtype=jnp.float32)
    w1bd = jnp.kron(eye4, W1.T.astype(jnp.float32)).astype(jnp.bfloat16)
    w2bd = (jnp.kron(eye4, W2.T.astype(jnp.float32))
            * (2.0 ** -12)).astype(jnp.bfloat16)
    b1t = jnp.tile(b1, PACK).reshape(1, DP)
    b2t = jnp.tile(b2, PACK).reshape(1, DP) * (2.0 ** -12)
    h4 = _edge_mlp(rbf.reshape(EP, DP), w1bd, b1t, w2bd, b2t)
    h = h4.reshape(E, D)
    return _sc_kernel(xp, h, src, dst)


# speedup vs baseline: 1.0167x; 1.0167x over previous
import functools

import numpy as np

import jax
import jax.numpy as jnp
from jax import lax
from jax.experimental import pallas as pl
from jax.experimental.pallas import tpu as pltpu
from jax.experimental.pallas import tpu_sc as plsc

N = 50000
E = 800000
D = 64
PACK = 4
EP = E // PACK
DP = D * PACK
BM = 1000

HALF = N // 2
ACC_ROWS = 25088
CHUNK = 128
NCHUNK = E // CHUNK
NSUB = 16
CPW = 392
NCPAD = NSUB * CPW


_LP = (1.472065010887924e-06, 0.9998476974962351, -0.49737321615793884,
       0.3157473167579205, -0.19035433673298097, 0.08269123711134978,
       -0.017414077524237504)


def _mlp_body(rbf_ref, w1_ref, b1_ref, w2_ref, b2_ref, out_ref):
    a = jnp.dot(rbf_ref[...].astype(jnp.bfloat16), w1_ref[...],
                preferred_element_type=jnp.float32)
    a = a + b1_ref[...]
    u = jnp.exp(-0.5 * jnp.abs(a))
    p = _LP[6]
    for k in (5, 4, 3, 2, 1, 0):
        p = p * u + _LP[k]
    a = jnp.maximum(a, 0.0) + 2.0 * p
    o = jnp.dot(a.astype(jnp.bfloat16), w2_ref[...],
                preferred_element_type=jnp.float32)
    out_ref[...] = o + b2_ref[...]


def _edge_mlp(rbf4, w1bd, b1t, w2bd, b2t):
    return pl.pallas_call(
        _mlp_body,
        grid=(EP // BM,),
        in_specs=[
            pl.BlockSpec((BM, DP), lambda i: (i, 0)),
            pl.BlockSpec((DP, DP), lambda i: (0, 0)),
            pl.BlockSpec((1, DP), lambda i: (0, 0)),
            pl.BlockSpec((DP, DP), lambda i: (0, 0)),
            pl.BlockSpec((1, DP), lambda i: (0, 0)),
        ],
        out_specs=pl.BlockSpec((BM, DP), lambda i: (i, 0)),
        out_shape=jax.ShapeDtypeStruct((EP, DP), jnp.float32),
    )(rbf4, w1bd, b1t, w2bd, b2t)


def _mul_rows(h_ref, x_ref):
    def _row(r, carry):
        for g in range(2):
            v = x_ref[r, pl.ds(g * 16, 16)]
            a = (v >> 16).astype(jnp.float32)
            b = ((v << 16) >> 16).astype(jnp.float32)
            s0 = pl.ds(g * 32, 16)
            s1 = pl.ds(g * 32 + 16, 16)
            h_ref[r, s0] = h_ref[r, s0] * b
            h_ref[r, s1] = h_ref[r, s1] * a
        return carry
    lax.fori_loop(0, CHUNK, _row, 0)


def _sc_body(x_hbm, h_hbm, src_hbm, dst_hbm, out_hbm,
             src_sv, dst_sv, x0, x1, h0, h1, acc,
             sga, sgb, sha, shb):
    c = lax.axis_index("c")
    s = lax.axis_index("s")
    w = s
    lo = c * HALF

    def _zrow(i, carry):
        for c4 in range(4):
            h0[i, pl.ds(c4 * 16, 16)] = jnp.zeros((16,), jnp.float32)
        return carry
    lax.fori_loop(0, CHUNK, _zrow, 0)
    zbase = s * (ACC_ROWS // NSUB)
    for k in range(ACC_ROWS // NSUB // CHUNK):
        pltpu.sync_copy(h0.at[pl.ds(0, CHUNK)],
                        acc.at[pl.ds(zbase + k * CHUNK, CHUNK)])
    pltpu.sync_copy(h0.at[pl.ds(0, 32)], acc.at[pl.ds(zbase + 1536, 32)])
    plsc.subcore_barrier()

    def _pair(jj, carry):
        sup = jj % 4
        base_t = w * CPW + jj * 2

        @pl.when(sup == 0)
        def _load_super():
            pltpu.sync_copy(src_hbm.at[pl.ds(base_t, 8)], src_sv)
            pltpu.sync_copy(dst_hbm.at[pl.ds(base_t, 8)], dst_sv)
            for r in range(8):
                lo_r = jnp.where(base_t + r < NCHUNK, lo, N + D)
                for c8 in range(8):
                    sl = pl.ds(c8 * 16, 16)
                    dv = dst_sv[r, sl]
                    m = (dv >= lo_r) & (dv < lo_r + HALF)
                    dst_sv[r, sl] = jnp.where(m, dv - lo_r, HALF)

        ra = sup * 2
        ta = jnp.minimum(base_t, NCHUNK - 1)
        tb = jnp.minimum(base_t + 1, NCHUNK - 1)
        cpxa = pltpu.async_copy(x_hbm.at[src_sv.at[ra]], x0, sga)
        cpha = pltpu.async_copy(h_hbm.at[pl.ds(ta * CHUNK, CHUNK)], h0, sha)
        cpxb = pltpu.async_copy(x_hbm.at[src_sv.at[ra + 1]], x1, sgb)
        cphb = pltpu.async_copy(h_hbm.at[pl.ds(tb * CHUNK, CHUNK)], h1, shb)
        cpha.wait()
        cpxa.wait()
        _mul_rows(h0, x0)
        pltpu.sync_copy(h0, acc.at[dst_sv.at[ra]], add=True)
        cphb.wait()
        cpxb.wait()
        _mul_rows(h1, x1)
        pltpu.sync_copy(h1, acc.at[dst_sv.at[ra + 1]], add=True)
        return carry
    lax.fori_loop(0, CPW // 2, _pair, 0)

    plsc.subcore_barrier()
    obase = lo + s * 1560
    pltpu.sync_copy(acc.at[pl.ds(s * 1560, 1560)],
                    out_hbm.at[pl.ds(obase, 1560)])

    @pl.when(s == NSUB - 1)
    def _tail():
        pltpu.sync_copy(acc.at[pl.ds(24960, 40)],
                        out_hbm.at[pl.ds(lo + 24960, 40)])


_sc_kernel = functools.partial(
    pl.kernel,
    mesh=plsc.VectorSubcoreMesh(core_axis_name="c", subcore_axis_name="s"),
    compiler_params=pltpu.CompilerParams(use_tc_tiling_on_sc=False),
    out_type=jax.ShapeDtypeStruct((N, D), jnp.float32),
    scratch_types=[
        pltpu.VMEM((8, 128), jnp.int32),
        pltpu.VMEM((8, 128), jnp.int32),
        pltpu.VMEM((CHUNK, D // 2), jnp.int32),
        pltpu.VMEM((CHUNK, D // 2), jnp.int32),
        pltpu.VMEM((CHUNK, D), jnp.float32),
        pltpu.VMEM((CHUNK, D), jnp.float32),
        pltpu.VMEM_SHARED((ACC_ROWS, D), jnp.float32),
        pltpu.SemaphoreType.DMA,
        pltpu.SemaphoreType.DMA,
        pltpu.SemaphoreType.DMA,
        pltpu.SemaphoreType.DMA,
    ],
)(_sc_body)


def kernel(x, rbf, edge_index, W1, b1, W2, b2):
    src = edge_index[0].astype(jnp.int32).reshape(NCHUNK, 128)
    dst = edge_index[1].astype(jnp.int32).reshape(NCHUNK, 128)
    pad = ((0, NCPAD - NCHUNK), (0, 0))
    src = jnp.pad(src, pad)
    dst = jnp.pad(dst, pad)
    xq = jnp.clip(jnp.round(x * 4096.0), -32768.0, 32767.0).astype(jnp.int16)
    xp = lax.bitcast_convert_type(
        xq.reshape(N, 2, 2, 16).transpose(0, 1, 3, 2), jnp.int32)
    xp = xp.reshape(N, D // 2)
    eye4 = jnp.eye(PACK, dtype=jnp.float32)
    w1bd = jnp.kron(eye4, W1.T.astype(jnp.float32)).astype(jnp.bfloat16)
    w2bd = (jnp.kron(eye4, W2.T.astype(jnp.float32))
            * (2.0 ** -12)).astype(jnp.bfloat16)
    b1t = jnp.tile(b1, PACK).reshape(1, DP)
    b2t = jnp.tile(b2, PACK).reshape(1, DP) * (2.0 ** -12)
    h4 = _edge_mlp(rbf.reshape(EP, DP), w1bd, b1t, w2bd, b2t)
    h = h4.reshape(E, D)
    return _sc_kernel(xp, h, src, dst)


# speedup vs baseline: 1.0472x; 1.0299x over previous
import functools

import numpy as np

import jax
import jax.numpy as jnp
from jax import lax
from jax.experimental import pallas as pl
from jax.experimental.pallas import tpu as pltpu
from jax.experimental.pallas import tpu_sc as plsc

N = 50000
E = 800000
D = 64
PACK = 4
EP = E // PACK
DP = D * PACK
BM = 2000

HALF = N // 2
ACC_ROWS = 25088
CHUNK = 128
NCHUNK = E // CHUNK
NSUB = 16
CPW = 392
NCPAD = NSUB * CPW


_LP = (1.472065010887924e-06, 0.9998476974962351, -0.49737321615793884,
       0.3157473167579205, -0.19035433673298097, 0.08269123711134978,
       -0.017414077524237504)


def _mlp_body(rbf_ref, w1_ref, b1_ref, w2_ref, b2_ref, out_ref):
    a = jnp.dot(rbf_ref[...].astype(jnp.bfloat16), w1_ref[...],
                preferred_element_type=jnp.float32)
    a = a + b1_ref[...]
    u = jnp.exp(-0.5 * jnp.abs(a))
    p = _LP[6]
    for k in (5, 4, 3, 2, 1, 0):
        p = p * u + _LP[k]
    a = jnp.maximum(a, 0.0) + 2.0 * p
    o = jnp.dot(a.astype(jnp.bfloat16), w2_ref[...],
                preferred_element_type=jnp.float32)
    out_ref[...] = o + b2_ref[...]


def _edge_mlp(rbf4, w1bd, b1t, w2bd, b2t):
    return pl.pallas_call(
        _mlp_body,
        grid=(EP // BM,),
        in_specs=[
            pl.BlockSpec((BM, DP), lambda i: (i, 0)),
            pl.BlockSpec((DP, DP), lambda i: (0, 0)),
            pl.BlockSpec((1, DP), lambda i: (0, 0)),
            pl.BlockSpec((DP, DP), lambda i: (0, 0)),
            pl.BlockSpec((1, DP), lambda i: (0, 0)),
        ],
        out_specs=pl.BlockSpec((BM, DP), lambda i: (i, 0)),
        out_shape=jax.ShapeDtypeStruct((EP, DP), jnp.float32),
    )(rbf4, w1bd, b1t, w2bd, b2t)


def _mul_rows(h_ref, x_ref):
    def _row(r, carry):
        for g in range(2):
            v = x_ref[r, pl.ds(g * 16, 16)]
            a = (v >> 16).astype(jnp.float32)
            b = ((v << 16) >> 16).astype(jnp.float32)
            s0 = pl.ds(g * 32, 16)
            s1 = pl.ds(g * 32 + 16, 16)
            h_ref[r, s0] = h_ref[r, s0] * b
            h_ref[r, s1] = h_ref[r, s1] * a
        return carry
    lax.fori_loop(0, CHUNK, _row, 0)


def _sc_body(x_hbm, h_hbm, src_hbm, dst_hbm, out_hbm,
             src_sv, dst_sv, x0, x1, h0, h1, acc,
             sga, sgb, sha, shb):
    c = lax.axis_index("c")
    s = lax.axis_index("s")
    w = s
    lo = c * HALF

    def _zrow(i, carry):
        for c4 in range(4):
            h0[i, pl.ds(c4 * 16, 16)] = jnp.zeros((16,), jnp.float32)
        return carry
    lax.fori_loop(0, CHUNK, _zrow, 0)
    zbase = s * (ACC_ROWS // NSUB)
    for k in range(ACC_ROWS // NSUB // CHUNK):
        pltpu.sync_copy(h0.at[pl.ds(0, CHUNK)],
                        acc.at[pl.ds(zbase + k * CHUNK, CHUNK)])
    pltpu.sync_copy(h0.at[pl.ds(0, 32)], acc.at[pl.ds(zbase + 1536, 32)])
    plsc.subcore_barrier()

    def _pair(jj, carry):
        sup = jj % 4
        base_t = w * CPW + jj * 2

        @pl.when(sup == 0)
        def _load_super():
            pltpu.sync_copy(src_hbm.at[pl.ds(base_t, 8)], src_sv)
            pltpu.sync_copy(dst_hbm.at[pl.ds(base_t, 8)], dst_sv)
            for r in range(8):
                lo_r = jnp.where(base_t + r < NCHUNK, lo, N + D)
                for c8 in range(8):
                    sl = pl.ds(c8 * 16, 16)
                    dv = dst_sv[r, sl]
                    m = (dv >= lo_r) & (dv < lo_r + HALF)
                    dst_sv[r, sl] = jnp.where(m, dv - lo_r, HALF)

        ra = sup * 2
        ta = jnp.minimum(base_t, NCHUNK - 1)
        tb = jnp.minimum(base_t + 1, NCHUNK - 1)
        cpxa = pltpu.async_copy(x_hbm.at[src_sv.at[ra]], x0, sga)
        cpha = pltpu.async_copy(h_hbm.at[pl.ds(ta * CHUNK, CHUNK)], h0, sha)
        cpxb = pltpu.async_copy(x_hbm.at[src_sv.at[ra + 1]], x1, sgb)
        cphb = pltpu.async_copy(h_hbm.at[pl.ds(tb * CHUNK, CHUNK)], h1, shb)
        cpha.wait()
        cpxa.wait()
        _mul_rows(h0, x0)
        pltpu.sync_copy(h0, acc.at[dst_sv.at[ra]], add=True)
        cphb.wait()
        cpxb.wait()
        _mul_rows(h1, x1)
        pltpu.sync_copy(h1, acc.at[dst_sv.at[ra + 1]], add=True)
        return carry
    lax.fori_loop(0, CPW // 2, _pair, 0)

    plsc.subcore_barrier()
    obase = lo + s * 1560
    pltpu.sync_copy(acc.at[pl.ds(s * 1560, 1560)],
                    out_hbm.at[pl.ds(obase, 1560)])

    @pl.when(s == NSUB - 1)
    def _tail():
        pltpu.sync_copy(acc.at[pl.ds(24960, 40)],
                        out_hbm.at[pl.ds(lo + 24960, 40)])


_sc_kernel = functools.partial(
    pl.kernel,
    mesh=plsc.VectorSubcoreMesh(core_axis_name="c", subcore_axis_name="s"),
    compiler_params=pltpu.CompilerParams(use_tc_tiling_on_sc=False),
    out_type=jax.ShapeDtypeStruct((N, D), jnp.float32),
    scratch_types=[
        pltpu.VMEM((8, 128), jnp.int32),
        pltpu.VMEM((8, 128), jnp.int32),
        pltpu.VMEM((CHUNK, D // 2), jnp.int32),
        pltpu.VMEM((CHUNK, D // 2), jnp.int32),
        pltpu.VMEM((CHUNK, D), jnp.float32),
        pltpu.VMEM((CHUNK, D), jnp.float32),
        pltpu.VMEM_SHARED((ACC_ROWS, D), jnp.float32),
        pltpu.SemaphoreType.DMA,
        pltpu.SemaphoreType.DMA,
        pltpu.SemaphoreType.DMA,
        pltpu.SemaphoreType.DMA,
    ],
)(_sc_body)


def kernel(x, rbf, edge_index, W1, b1, W2, b2):
    src = edge_index[0].astype(jnp.int32).reshape(NCHUNK, 128)
    dst = edge_index[1].astype(jnp.int32).reshape(NCHUNK, 128)
    pad = ((0, NCPAD - NCHUNK), (0, 0))
    src = jnp.pad(src, pad)
    dst = jnp.pad(dst, pad)
    xq = jnp.clip(jnp.round(x * 4096.0), -32768.0, 32767.0).astype(jnp.int16)
    xp = lax.bitcast_convert_type(
        xq.reshape(N, 2, 2, 16).transpose(0, 1, 3, 2), jnp.int32)
    xp = xp.reshape(N, D // 2)
    eye4 = jnp.eye(PACK, dtype=jnp.float32)
    w1bd = jnp.kron(eye4, W1.T.astype(jnp.float32)).astype(jnp.bfloat16)
    w2bd = (jnp.kron(eye4, W2.T.astype(jnp.float32))
            * (2.0 ** -12)).astype(jnp.bfloat16)
    b1t = jnp.tile(b1, PACK).reshape(1, DP)
    b2t = jnp.tile(b2, PACK).reshape(1, DP) * (2.0 ** -12)
    h4 = _edge_mlp(rbf.reshape(EP, DP), w1bd, b1t, w2bd, b2t)
    h = h4.reshape(E, D)
    return _sc_kernel(xp, h, src, dst)


# speedup vs baseline: 1.0635x; 1.0156x over previous
import functools

import numpy as np

import jax
import jax.numpy as jnp
from jax import lax
from jax.experimental import pallas as pl
from jax.experimental.pallas import tpu as pltpu
from jax.experimental.pallas import tpu_sc as plsc

N = 50000
E = 800000
D = 64
PACK = 4
EP = E // PACK
DP = D * PACK
BM = 4000

HALF = N // 2
ACC_ROWS = 25088
CHUNK = 128
NCHUNK = E // CHUNK
NSUB = 16
CPW = 392
NCPAD = NSUB * CPW


_LP = (1.472065010887924e-06, 0.9998476974962351, -0.49737321615793884,
       0.3157473167579205, -0.19035433673298097, 0.08269123711134978,
       -0.017414077524237504)


def _mlp_body(rbf_ref, w1_ref, b1_ref, w2_ref, b2_ref, out_ref):
    a = jnp.dot(rbf_ref[...].astype(jnp.bfloat16), w1_ref[...],
                preferred_element_type=jnp.float32)
    a = a + b1_ref[...]
    u = jnp.exp(-0.5 * jnp.abs(a))
    p = _LP[6]
    for k in (5, 4, 3, 2, 1, 0):
        p = p * u + _LP[k]
    a = jnp.maximum(a, 0.0) + 2.0 * p
    o = jnp.dot(a.astype(jnp.bfloat16), w2_ref[...],
                preferred_element_type=jnp.float32)
    out_ref[...] = o + b2_ref[...]


def _edge_mlp(rbf4, w1bd, b1t, w2bd, b2t):
    return pl.pallas_call(
        _mlp_body,
        grid=(EP // BM,),
        in_specs=[
            pl.BlockSpec((BM, DP), lambda i: (i, 0)),
            pl.BlockSpec((DP, DP), lambda i: (0, 0)),
            pl.BlockSpec((1, DP), lambda i: (0, 0)),
            pl.BlockSpec((DP, DP), lambda i: (0, 0)),
            pl.BlockSpec((1, DP), lambda i: (0, 0)),
        ],
        out_specs=pl.BlockSpec((BM, DP), lambda i: (i, 0)),
        out_shape=jax.ShapeDtypeStruct((EP, DP), jnp.float32),
    )(rbf4, w1bd, b1t, w2bd, b2t)


def _mul_rows(h_ref, x_ref):
    def _row(r, carry):
        for g in range(2):
            v = x_ref[r, pl.ds(g * 16, 16)]
            a = (v >> 16).astype(jnp.float32)
            b = ((v << 16) >> 16).astype(jnp.float32)
            s0 = pl.ds(g * 32, 16)
            s1 = pl.ds(g * 32 + 16, 16)
            h_ref[r, s0] = h_ref[r, s0] * b
            h_ref[r, s1] = h_ref[r, s1] * a
        return carry
    lax.fori_loop(0, CHUNK, _row, 0)


def _sc_body(x_hbm, h_hbm, src_hbm, dst_hbm, out_hbm,
             src_sv, dst_sv, x0, x1, h0, h1, acc,
             sga, sgb, sha, shb):
    c = lax.axis_index("c")
    s = lax.axis_index("s")
    w = s
    lo = c * HALF

    def _zrow(i, carry):
        for c4 in range(4):
            h0[i, pl.ds(c4 * 16, 16)] = jnp.zeros((16,), jnp.float32)
        return carry
    lax.fori_loop(0, CHUNK, _zrow, 0)
    zbase = s * (ACC_ROWS // NSUB)
    for k in range(ACC_ROWS // NSUB // CHUNK):
        pltpu.sync_copy(h0.at[pl.ds(0, CHUNK)],
                        acc.at[pl.ds(zbase + k * CHUNK, CHUNK)])
    pltpu.sync_copy(h0.at[pl.ds(0, 32)], acc.at[pl.ds(zbase + 1536, 32)])
    plsc.subcore_barrier()

    def _pair(jj, carry):
        sup = jj % 4
        base_t = w * CPW + jj * 2

        @pl.when(sup == 0)
        def _load_super():
            pltpu.sync_copy(src_hbm.at[pl.ds(base_t, 8)], src_sv)
            pltpu.sync_copy(dst_hbm.at[pl.ds(base_t, 8)], dst_sv)
            for r in range(8):
                lo_r = jnp.where(base_t + r < NCHUNK, lo, N + D)
                for c8 in range(8):
                    sl = pl.ds(c8 * 16, 16)
                    dv = dst_sv[r, sl]
                    m = (dv >= lo_r) & (dv < lo_r + HALF)
                    dst_sv[r, sl] = jnp.where(m, dv - lo_r, HALF)

        ra = sup * 2
        ta = jnp.minimum(base_t, NCHUNK - 1)
        tb = jnp.minimum(base_t + 1, NCHUNK - 1)
        cpxa = pltpu.async_copy(x_hbm.at[src_sv.at[ra]], x0, sga)
        cpha = pltpu.async_copy(h_hbm.at[pl.ds(ta * CHUNK, CHUNK)], h0, sha)
        cpxb = pltpu.async_copy(x_hbm.at[src_sv.at[ra + 1]], x1, sgb)
        cphb = pltpu.async_copy(h_hbm.at[pl.ds(tb * CHUNK, CHUNK)], h1, shb)
        cpha.wait()
        cpxa.wait()
        _mul_rows(h0, x0)
        pltpu.sync_copy(h0, acc.at[dst_sv.at[ra]], add=True)
        cphb.wait()
        cpxb.wait()
        _mul_rows(h1, x1)
        pltpu.sync_copy(h1, acc.at[dst_sv.at[ra + 1]], add=True)
        return carry
    lax.fori_loop(0, CPW // 2, _pair, 0)

    plsc.subcore_barrier()
    obase = lo + s * 1560
    pltpu.sync_copy(acc.at[pl.ds(s * 1560, 1560)],
                    out_hbm.at[pl.ds(obase, 1560)])

    @pl.when(s == NSUB - 1)
    def _tail():
        pltpu.sync_copy(acc.at[pl.ds(24960, 40)],
                        out_hbm.at[pl.ds(lo + 24960, 40)])


_sc_kernel = functools.partial(
    pl.kernel,
    mesh=plsc.VectorSubcoreMesh(core_axis_name="c", subcore_axis_name="s"),
    compiler_params=pltpu.CompilerParams(use_tc_tiling_on_sc=False),
    out_type=jax.ShapeDtypeStruct((N, D), jnp.float32),
    scratch_types=[
        pltpu.VMEM((8, 128), jnp.int32),
        pltpu.VMEM((8, 128), jnp.int32),
        pltpu.VMEM((CHUNK, D // 2), jnp.int32),
        pltpu.VMEM((CHUNK, D // 2), jnp.int32),
        pltpu.VMEM((CHUNK, D), jnp.float32),
        pltpu.VMEM((CHUNK, D), jnp.float32),
        pltpu.VMEM_SHARED((ACC_ROWS, D), jnp.float32),
        pltpu.SemaphoreType.DMA,
        pltpu.SemaphoreType.DMA,
        pltpu.SemaphoreType.DMA,
        pltpu.SemaphoreType.DMA,
    ],
)(_sc_body)


def kernel(x, rbf, edge_index, W1, b1, W2, b2):
    src = edge_index[0].astype(jnp.int32).reshape(NCHUNK, 128)
    dst = edge_index[1].astype(jnp.int32).reshape(NCHUNK, 128)
    pad = ((0, NCPAD - NCHUNK), (0, 0))
    src = jnp.pad(src, pad)
    dst = jnp.pad(dst, pad)
    xq = jnp.clip(jnp.round(x * 4096.0), -32768.0, 32767.0).astype(jnp.int16)
    xp = lax.bitcast_convert_type(
        xq.reshape(N, 2, 2, 16).transpose(0, 1, 3, 2), jnp.int32)
    xp = xp.reshape(N, D // 2)
    eye4 = jnp.eye(PACK, dtype=jnp.float32)
    w1bd = jnp.kron(eye4, W1.T.astype(jnp.float32)).astype(jnp.bfloat16)
    w2bd = (jnp.kron(eye4, W2.T.astype(jnp.float32))
            * (2.0 ** -12)).astype(jnp.bfloat16)
    b1t = jnp.tile(b1, PACK).reshape(1, DP)
    b2t = jnp.tile(b2, PACK).reshape(1, DP) * (2.0 ** -12)
    h4 = _edge_mlp(rbf.reshape(EP, DP), w1bd, b1t, w2bd, b2t)
    h = h4.reshape(E, D)
    return _sc_kernel(xp, h, src, dst)
